# Initial kernel scaffold; baseline (speedup 1.0000x reference)
import numpy as np
import jax
import jax.numpy as jnp
from jax.experimental import pallas as pl
from jax.experimental.pallas import tpu as pltpu

B = 1024
NJ = 14
COL = 14
CC = COL * COL


def _build_tables():
    radius = 4
    xk = np.arange(-radius, radius + 1)
    k = np.exp(-0.5 * xk.astype(np.float64) ** 2)
    k = (k / k.sum()).astype(np.float32)
    prof = np.zeros((COL, COL), np.float32)
    for c in range(COL):
        d = np.zeros(COL, np.float32)
        d[c] = 1.0
        p = np.pad(d, radius, mode='symmetric')
        for i in range(COL):
            prof[c, i] = np.dot(k, p[i:i + 2 * radius + 1])
    T = prof / prof.max(axis=1, keepdims=True)
    S2 = (T * T).sum(axis=1)
    ly = np.arange(CC) // COL
    lx = np.arange(CC) % COL
    TyE = T[:, ly]
    TxE = T[:, lx]
    return T, S2, TyE, TxE


_T_np, _S2_np, _TyE_np, _TxE_np = _build_tables()


def _loss_kernel(h_ref, osx_ref, osy_ref, tv_ref, tye_ref, txe_ref, s2_ref,
                 out_ref, acc_ref):
    i = pl.program_id(0)
    nblocks = pl.num_programs(0)
    RB = h_ref.shape[0]
    BB = RB // NJ

    hb = h_ref[...]
    blocksq = jnp.sum(hb * hb)

    r0 = hb[:, :COL]
    r0sq = jnp.sum(r0 * r0, axis=1, keepdims=True)

    vals = jnp.max(hb, axis=1, keepdims=True)
    lane = jax.lax.broadcasted_iota(jnp.int32, (RB, CC), 1)
    am = jnp.min(jnp.where(hb == vals, lane, CC), axis=1, keepdims=True)
    yC = am // COL
    xC = am - yC * COL

    tv = tv_ref[...]
    tx = tv[:, 0:1]
    ty = tv[:, 1:2]
    v0 = tv[:, 2:3]
    v1 = tv[:, 3:4]
    xi = jnp.clip((tx * COL).astype(jnp.int32), 0, COL - 1)
    yi = jnp.clip((ty * COL).astype(jnp.int32), 0, COL - 1)

    l14 = jax.lax.broadcasted_iota(jnp.int32, (RB, COL), 1)
    ohy = (yi == l14).astype(jnp.float32)
    ohx = (xi == l14).astype(jnp.float32)
    wy = jnp.dot(ohy, tye_ref[...], preferred_element_type=jnp.float32)
    wx = jnp.dot(ohx, txe_ref[...], preferred_element_type=jnp.float32)
    bil = jnp.sum(hb * wy * wx, axis=1, keepdims=True)

    s2 = s2_ref[...]
    s2y = jnp.sum(ohy * s2, axis=1, keepdims=True)
    s2x = jnp.sum(ohx * s2, axis=1, keepdims=True)
    tts = s2y * s2x

    vis = v0 == 1.0
    part1 = blocksq + jnp.sum(jnp.where(vis, tts - 2.0 * bil, -r0sq))

    sel = (lane == am).astype(jnp.float32)
    sel3 = sel.reshape(BB, NJ, CC)
    osx = osx_ref[...].reshape(BB, NJ, CC)
    osy = osy_ref[...].reshape(BB, NJ, CC)
    ox3 = jnp.sum(osx * sel3, axis=2)
    oy3 = jnp.sum(osy * sel3, axis=2)
    ox = ox3.reshape(RB, 1)
    oy = oy3.reshape(RB, 1)

    scale = 1.0 / COL
    mask = vals > 0.5
    xCf = xC.astype(jnp.float32)
    yCf = yC.astype(jnp.float32)
    x0 = jnp.where(mask, (ox + xCf) * scale, 0.0)
    x1 = jnp.where(mask, (oy + yCf) * scale, 0.0)
    d2a = (x0 - tx) * v0
    d2b = (x1 - ty) * v1
    part2 = jnp.sum(d2a * d2a + d2b * d2b)

    vsum = jnp.sum(v0 + v1)

    @pl.when(i == 0)
    def _():
        acc_ref[0] = 0.0
        acc_ref[1] = 0.0

    acc_ref[0] = acc_ref[0] + part1 + part2
    acc_ref[1] = acc_ref[1] + vsum

    @pl.when(i == nblocks - 1)
    def _():
        out_ref[0, 0] = acc_ref[0] / (acc_ref[1] * 0.5)


@jax.jit
def _run(os, h, t, v):
    h2 = h.reshape(B * NJ, CC)
    os4 = os.reshape(B, 2, NJ, CC)
    tv = jnp.concatenate([t, v], axis=-1).reshape(B * NJ, 4)

    tye = jnp.asarray(_TyE_np)
    txe = jnp.asarray(_TxE_np)
    s2 = jnp.asarray(_S2_np).reshape(1, COL)

    BB = 128
    RB = BB * NJ
    grid = (B // BB,)

    out = pl.pallas_call(
        _loss_kernel,
        grid=grid,
        in_specs=[
            pl.BlockSpec((RB, CC), lambda i: (i, 0)),
            pl.BlockSpec((BB, 1, NJ, CC), lambda i: (i, 0, 0, 0)),
            pl.BlockSpec((BB, 1, NJ, CC), lambda i: (i, 1, 0, 0)),
            pl.BlockSpec((RB, 4), lambda i: (i, 0)),
            pl.BlockSpec((COL, CC), lambda i: (0, 0)),
            pl.BlockSpec((COL, CC), lambda i: (0, 0)),
            pl.BlockSpec((1, COL), lambda i: (0, 0)),
        ],
        out_specs=pl.BlockSpec((1, 1), lambda i: (0, 0)),
        out_shape=jax.ShapeDtypeStruct((1, 1), jnp.float32),
        scratch_shapes=[pltpu.SMEM((2,), jnp.float32)],
        compiler_params=pltpu.CompilerParams(
            dimension_semantics=("arbitrary",),
        ),
    )(h2, os4, os4, tv, tye, txe, s2)
    return out[0, 0]


def kernel(os, h, op, t, v):
    return _run(os, h, t, v)


# speedup vs baseline: 1.9235x; 1.9235x over previous
import numpy as np
import jax
import jax.numpy as jnp
from jax.experimental import pallas as pl
from jax.experimental.pallas import tpu as pltpu

B = 1024
NJ = 14
COL = 14
CC = COL * COL


def _build_tables():
    radius = 4
    xk = np.arange(-radius, radius + 1)
    k = np.exp(-0.5 * xk.astype(np.float64) ** 2)
    k = (k / k.sum()).astype(np.float32)
    prof = np.zeros((COL, COL), np.float32)
    for c in range(COL):
        d = np.zeros(COL, np.float32)
        d[c] = 1.0
        p = np.pad(d, radius, mode='symmetric')
        for i in range(COL):
            prof[c, i] = np.dot(k, p[i:i + 2 * radius + 1])
    T = prof / prof.max(axis=1, keepdims=True)
    S2 = (T * T).sum(axis=1)
    ly = np.arange(CC) // COL
    lx = np.arange(CC) % COL
    TyE = T[:, ly]
    TxE = T[:, lx]
    return T, S2, TyE, TxE


_T_np, _S2_np, _TyE_np, _TxE_np = _build_tables()


def _loss_kernel(h_ref, osx_ref, osy_ref, tv_ref, tye_ref, txe_ref, s2_ref,
                 out_ref, acc_ref):
    i = pl.program_id(0)
    nblocks = pl.num_programs(0)
    RB = h_ref.shape[0]
    BB = RB // NJ

    hb = h_ref[...]
    blocksq = jnp.sum(hb * hb)

    r0 = hb[:, :COL]
    r0sq = jnp.sum(r0 * r0, axis=1, keepdims=True)

    vals = jnp.max(hb, axis=1, keepdims=True)
    lane = jax.lax.broadcasted_iota(jnp.int32, (RB, CC), 1)
    am = jnp.min(jnp.where(hb == vals, lane, CC), axis=1, keepdims=True)
    yC = am // COL
    xC = am - yC * COL

    tv = tv_ref[...]
    tx = tv[:, 0:1]
    ty = tv[:, 1:2]
    v0 = tv[:, 2:3]
    v1 = tv[:, 3:4]
    xi = jnp.clip((tx * COL).astype(jnp.int32), 0, COL - 1)
    yi = jnp.clip((ty * COL).astype(jnp.int32), 0, COL - 1)

    l14 = jax.lax.broadcasted_iota(jnp.int32, (RB, COL), 1)
    ohy = (yi == l14).astype(jnp.float32)
    ohx = (xi == l14).astype(jnp.float32)
    wy = jnp.dot(ohy, tye_ref[...], preferred_element_type=jnp.float32)
    wx = jnp.dot(ohx, txe_ref[...], preferred_element_type=jnp.float32)
    bil = jnp.sum(hb * wy * wx, axis=1, keepdims=True)

    s2 = s2_ref[...]
    s2y = jnp.sum(ohy * s2, axis=1, keepdims=True)
    s2x = jnp.sum(ohx * s2, axis=1, keepdims=True)
    tts = s2y * s2x

    vis = v0 == 1.0
    part1 = blocksq + jnp.sum(jnp.where(vis, tts - 2.0 * bil, -r0sq))

    sel = (lane == am).astype(jnp.float32)
    sel3 = sel.reshape(BB, NJ, CC)
    osx = osx_ref[...].reshape(BB, NJ, CC)
    osy = osy_ref[...].reshape(BB, NJ, CC)
    ox3 = jnp.sum(osx * sel3, axis=2)
    oy3 = jnp.sum(osy * sel3, axis=2)
    ox = ox3.reshape(RB, 1)
    oy = oy3.reshape(RB, 1)

    scale = 1.0 / COL
    mask = vals > 0.5
    xCf = xC.astype(jnp.float32)
    yCf = yC.astype(jnp.float32)
    x0 = jnp.where(mask, (ox + xCf) * scale, 0.0)
    x1 = jnp.where(mask, (oy + yCf) * scale, 0.0)
    d2a = (x0 - tx) * v0
    d2b = (x1 - ty) * v1
    part2 = jnp.sum(d2a * d2a + d2b * d2b)

    vsum = jnp.sum(v0 + v1)

    @pl.when(i == 0)
    def _():
        acc_ref[0] = 0.0
        acc_ref[1] = 0.0

    acc_ref[0] = acc_ref[0] + part1 + part2
    acc_ref[1] = acc_ref[1] + vsum

    @pl.when(i == nblocks - 1)
    def _():
        out_ref[...] = jnp.broadcast_to(acc_ref[0] / (acc_ref[1] * 0.5), (1, 1))


@jax.jit
def _run(os, h, t, v):
    h2 = h.reshape(B * NJ, CC)
    os4 = os.reshape(B, 2, NJ, CC)
    tv = jnp.concatenate([t, v], axis=-1).reshape(B * NJ, 4)

    tye = jnp.asarray(_TyE_np)
    txe = jnp.asarray(_TxE_np)
    s2 = jnp.asarray(_S2_np).reshape(1, COL)

    BB = 128
    RB = BB * NJ
    grid = (B // BB,)

    out = pl.pallas_call(
        _loss_kernel,
        grid=grid,
        in_specs=[
            pl.BlockSpec((RB, CC), lambda i: (i, 0)),
            pl.BlockSpec((BB, 1, NJ, CC), lambda i: (i, 0, 0, 0)),
            pl.BlockSpec((BB, 1, NJ, CC), lambda i: (i, 1, 0, 0)),
            pl.BlockSpec((RB, 4), lambda i: (i, 0)),
            pl.BlockSpec((COL, CC), lambda i: (0, 0)),
            pl.BlockSpec((COL, CC), lambda i: (0, 0)),
            pl.BlockSpec((1, COL), lambda i: (0, 0)),
        ],
        out_specs=pl.BlockSpec((1, 1), lambda i: (0, 0)),
        out_shape=jax.ShapeDtypeStruct((1, 1), jnp.float32),
        scratch_shapes=[pltpu.SMEM((2,), jnp.float32)],
        compiler_params=pltpu.CompilerParams(
            dimension_semantics=("arbitrary",),
        ),
    )(h2, os4, os4, tv, tye, txe, s2)
    return out[0, 0]


def kernel(os, h, op, t, v):
    return _run(os, h, t, v)
